# baseline (speedup 1.0000x reference)
import functools

import jax
import jax.numpy as jnp
from jax import lax
from jax.experimental import pallas as pl
from jax.experimental.pallas import tpu as pltpu
from jax.experimental.pallas import tpu_sc as plsc

M = 16384
B = 1024
H = 256

_NC = 2
_NS = 16
_NW = _NC * _NS
_MP = M + _NW
_SPW = B // _NW


def _gather_body(ent_hbm, idx_hbm, out_hbm, idx_v, rows_v, sem):
    wid = lax.axis_index("s") * _NC + lax.axis_index("c")
    base = wid * _SPW
    pltpu.sync_copy(idx_hbm.at[pl.ds(base, _SPW)], idx_v)
    pltpu.async_copy(ent_hbm.at[idx_v], rows_v, sem).wait()
    pltpu.sync_copy(rows_v, out_hbm.at[pl.ds(base, _SPW)])


_sc_gather = functools.partial(
    pl.kernel,
    out_type=jax.ShapeDtypeStruct((B, H), jnp.float32),
    mesh=plsc.VectorSubcoreMesh(core_axis_name="c", subcore_axis_name="s"),
    scratch_types=[
        pltpu.VMEM((_SPW,), jnp.int32),
        pltpu.VMEM((_SPW, H), jnp.float32),
        pltpu.SemaphoreType.DMA,
    ],
)(_gather_body)


def _prep_body(h_ref, t_ref, wd_ref, we_ref, idxr_ref, idxc_ref,
               rmap_ref, hw_ref, qh_ref, mt_ref):
    h = h_ref[...]
    eqm = idxc_ref[...] == idxr_ref[...]
    jcol = lax.broadcasted_iota(jnp.int32, (B, B), 1)
    winner = jnp.max(jnp.where(eqm, jcol, -1), axis=1, keepdims=True)
    brow = lax.broadcasted_iota(jnp.int32, (B, 1), 0)
    trash = M + brow // _SPW
    rmap_ref[...] = jnp.where(winner == brow, idxc_ref[...], trash)
    hw_ref[...] = lax.dot_general(h, we_ref[...], (((1,), (1,)), ((), ())),
                                  preferred_element_type=jnp.float32)
    qh_ref[...] = lax.dot_general(h, wd_ref[...], (((1,), (1,)), ((), ())),
                                  preferred_element_type=jnp.float32)
    mt_ref[...] = jnp.mean(t_ref[...], axis=(0, 1), keepdims=True)


_tc_prep = pl.pallas_call(
    _prep_body,
    out_shape=(
        jax.ShapeDtypeStruct((B, 1), jnp.int32),
        jax.ShapeDtypeStruct((B, H), jnp.float32),
        jax.ShapeDtypeStruct((B, H), jnp.float32),
        jax.ShapeDtypeStruct((1, 1), jnp.float32),
    ),
)


def _update_body(e_ref, h_ref, qh_ref, wdb_ref, u_ref):
    e = e_ref[...]
    h = h_ref[...]
    logit = jnp.sum(e * qh_ref[...], axis=1, keepdims=True) + wdb_ref[0, 0]
    delta = jax.nn.sigmoid(logit)
    u = delta * e + (1.0 - delta) * h
    u_ref[...] = u * lax.rsqrt(jnp.sum(u * u, axis=1, keepdims=True))


_tc_update = pl.pallas_call(
    _update_body,
    out_shape=jax.ShapeDtypeStruct((B, H), jnp.float32),
)


def _scatter_body(u_hbm, t_hbm, rmap_hbm, ent_ref, dist_ref,
                  idx_v, u_v, t_v, sem):
    wid = lax.axis_index("s") * _NC + lax.axis_index("c")
    sbase = wid * _SPW
    cp1 = pltpu.async_copy(rmap_hbm.at[pl.ds(sbase, _SPW)], idx_v, sem)
    cp2 = pltpu.async_copy(u_hbm.at[pl.ds(sbase, _SPW)], u_v, sem)
    cp3 = pltpu.async_copy(t_hbm.at[pl.ds(sbase, _SPW)], t_v, sem)
    cp1.wait()
    cp2.wait()
    cp3.wait()
    sc1 = pltpu.async_copy(u_v, ent_ref.at[idx_v], sem)
    sc2 = pltpu.async_copy(t_v, dist_ref.at[idx_v], sem)
    sc1.wait()
    sc2.wait()


_sc_scatter = functools.partial(
    pl.kernel,
    out_type=(),
    mesh=plsc.VectorSubcoreMesh(core_axis_name="c", subcore_axis_name="s"),
    scratch_types=[
        pltpu.VMEM((_SPW,), jnp.int32),
        pltpu.VMEM((_SPW, H), jnp.float32),
        pltpu.VMEM((_SPW,), jnp.float32),
        pltpu.SemaphoreType.DMA,
    ],
)(_scatter_body)


_MBLK = 4096


def _score_body(hw_ref, e_ref, d_ref, mt_ref, w_ref, b_ref, o_ref):
    s = lax.dot_general(hw_ref[...].astype(jnp.bfloat16),
                        e_ref[...].astype(jnp.bfloat16),
                        (((1,), (1,)), ((), ())),
                        preferred_element_type=jnp.float32)
    term = (d_ref[...] - mt_ref[0, 0]) * w_ref[0, 0] + b_ref[0, 0]
    o_ref[...] = s + term


_tc_score = pl.pallas_call(
    _score_body,
    grid=(M // _MBLK,),
    in_specs=[
        pl.BlockSpec((B, H), lambda i: (0, 0)),
        pl.BlockSpec((_MBLK, H), lambda i: (i, 0)),
        pl.BlockSpec((1, _MBLK), lambda i: (0, i)),
        pl.BlockSpec((1, 1), lambda i: (0, 0)),
        pl.BlockSpec((1, 1), lambda i: (0, 0)),
        pl.BlockSpec((1, 1), lambda i: (0, 0)),
    ],
    out_specs=pl.BlockSpec((B, _MBLK), lambda i: (0, i)),
    out_shape=jax.ShapeDtypeStruct((B, M), jnp.float32),
)


def kernel(entities, dist_features, h, t, W_delta_w, W_delta_b, W_entity_w,
           W_entity_b, w_dist_w, w_dist_b, idx):
    idx = idx.astype(jnp.int32)
    ent_pad = jnp.concatenate(
        [entities, jnp.zeros((_NW, H), jnp.float32)], axis=0)
    dist_pad = jnp.concatenate(
        [dist_features.reshape(M), jnp.zeros((_NW,), jnp.float32)], axis=0)
    e = _sc_gather(ent_pad, idx)
    rmap, hW, qh, mean_t = _tc_prep(
        h, t.reshape(1, B), W_delta_w, W_entity_w,
        idx.reshape(1, B), idx.reshape(B, 1))
    u = _tc_update(e, h, qh, W_delta_b.reshape(1, 1))
    ent_ref = jax.new_ref(ent_pad)
    dist_ref = jax.new_ref(dist_pad)
    _sc_scatter(u, t.reshape(B), rmap.reshape(B), ent_ref, dist_ref)
    new_ent = ent_ref[...]
    new_dist = dist_ref[...]
    bias = (W_entity_b + w_dist_b).reshape(1, 1)
    return _tc_score(hW, new_ent, new_dist.reshape(1, _MP), mean_t,
                     w_dist_w.reshape(1, 1), bias)


# speedup vs baseline: 1.0702x; 1.0702x over previous
import functools

import jax
import jax.numpy as jnp
from jax import lax
from jax.experimental import pallas as pl
from jax.experimental.pallas import tpu as pltpu
from jax.experimental.pallas import tpu_sc as plsc

M = 16384
B = 1024
H = 256

_NC = 2
_NS = 16
_NW = _NC * _NS
_SPW = B // _NW


def _gather_body(ent_hbm, idx_hbm, out_hbm, idx_v, rows_v, sem):
    wid = lax.axis_index("s") * _NC + lax.axis_index("c")
    base = wid * _SPW
    pltpu.sync_copy(idx_hbm.at[pl.ds(base, _SPW)], idx_v)
    pltpu.async_copy(ent_hbm.at[idx_v], rows_v, sem).wait()
    pltpu.sync_copy(rows_v, out_hbm.at[pl.ds(base, _SPW)])


_sc_gather = functools.partial(
    pl.kernel,
    out_type=jax.ShapeDtypeStruct((B, H), jnp.float32),
    mesh=plsc.VectorSubcoreMesh(core_axis_name="c", subcore_axis_name="s"),
    scratch_types=[
        pltpu.VMEM((_SPW,), jnp.int32),
        pltpu.VMEM((_SPW, H), jnp.float32),
        pltpu.SemaphoreType.DMA,
    ],
)(_gather_body)


def _update_body(e_ref, h_ref, t_ref, wd_ref, wdb_ref, we_ref, idxr_ref,
                 idxc_ref, upd_ref, tfin_ref, hw_ref, mt_ref):
    e = e_ref[...]
    h = h_ref[...]
    eW = jnp.dot(e, wd_ref[...], preferred_element_type=jnp.float32)
    logit = jnp.sum(eW * h, axis=1, keepdims=True) + wdb_ref[0, 0]
    delta = jax.nn.sigmoid(logit)
    u = delta * e + (1.0 - delta) * h
    u = u * lax.rsqrt(jnp.sum(u * u, axis=1, keepdims=True))
    eqm = idxc_ref[...] == idxr_ref[...]
    jcol = lax.broadcasted_iota(jnp.int32, (B, B), 1)
    winner = jnp.max(jnp.where(eqm, jcol, -1), axis=1, keepdims=True)
    P = (jcol == winner).astype(jnp.float32)
    upd_ref[...] = jnp.dot(P, u, preferred_element_type=jnp.float32)
    tfin_ref[...] = lax.dot_general(t_ref[...], P, (((1,), (1,)), ((), ())),
                                    preferred_element_type=jnp.float32)
    hw_ref[...] = lax.dot_general(h, we_ref[...], (((1,), (1,)), ((), ())),
                                  preferred_element_type=jnp.float32)
    mt_ref[...] = jnp.mean(t_ref[...], axis=(0, 1), keepdims=True)


_tc_update = pl.pallas_call(
    _update_body,
    out_shape=(
        jax.ShapeDtypeStruct((B, H), jnp.float32),
        jax.ShapeDtypeStruct((1, B), jnp.float32),
        jax.ShapeDtypeStruct((B, H), jnp.float32),
        jax.ShapeDtypeStruct((1, 1), jnp.float32),
    ),
)


def _scatter_body(upd_hbm, tfin_hbm, idx_hbm, ent_ref, dist_ref,
                  idx_v, upd_v, t_v, sem):
    wid = lax.axis_index("s") * _NC + lax.axis_index("c")
    sbase = wid * _SPW
    cp1 = pltpu.async_copy(idx_hbm.at[pl.ds(sbase, _SPW)], idx_v, sem)
    cp2 = pltpu.async_copy(upd_hbm.at[pl.ds(sbase, _SPW)], upd_v, sem)
    cp3 = pltpu.async_copy(tfin_hbm.at[pl.ds(sbase, _SPW)], t_v, sem)
    cp1.wait()
    cp2.wait()
    cp3.wait()
    sc1 = pltpu.async_copy(upd_v, ent_ref.at[idx_v], sem)
    sc2 = pltpu.async_copy(t_v, dist_ref.at[idx_v], sem)
    sc1.wait()
    sc2.wait()


_sc_scatter = functools.partial(
    pl.kernel,
    out_type=(),
    mesh=plsc.VectorSubcoreMesh(core_axis_name="c", subcore_axis_name="s"),
    scratch_types=[
        pltpu.VMEM((_SPW,), jnp.int32),
        pltpu.VMEM((_SPW, H), jnp.float32),
        pltpu.VMEM((_SPW,), jnp.float32),
        pltpu.SemaphoreType.DMA,
    ],
)(_scatter_body)


_MBLK = 4096


def _score_body(hw_ref, e_ref, d_ref, mt_ref, w_ref, b_ref, o_ref):
    s = lax.dot_general(hw_ref[...].astype(jnp.bfloat16),
                        e_ref[...].astype(jnp.bfloat16),
                        (((1,), (1,)), ((), ())),
                        preferred_element_type=jnp.float32)
    term = (d_ref[...] - mt_ref[0, 0]) * w_ref[0, 0] + b_ref[0, 0]
    o_ref[...] = s + term


_tc_score = pl.pallas_call(
    _score_body,
    grid=(M // _MBLK,),
    in_specs=[
        pl.BlockSpec((B, H), lambda i: (0, 0)),
        pl.BlockSpec((_MBLK, H), lambda i: (i, 0)),
        pl.BlockSpec((1, _MBLK), lambda i: (0, i)),
        pl.BlockSpec((1, 1), lambda i: (0, 0)),
        pl.BlockSpec((1, 1), lambda i: (0, 0)),
        pl.BlockSpec((1, 1), lambda i: (0, 0)),
    ],
    out_specs=pl.BlockSpec((B, _MBLK), lambda i: (0, i)),
    out_shape=jax.ShapeDtypeStruct((B, M), jnp.float32),
)


def kernel(entities, dist_features, h, t, W_delta_w, W_delta_b, W_entity_w,
           W_entity_b, w_dist_w, w_dist_b, idx):
    idx = idx.astype(jnp.int32)
    e = _sc_gather(entities, idx)
    upd, tfin, hW, mean_t = _tc_update(
        e, h, t.reshape(1, B), W_delta_w, W_delta_b.reshape(1, 1), W_entity_w,
        idx.reshape(1, B), idx.reshape(B, 1))
    ent_ref = jax.new_ref(entities)
    dist_ref = jax.new_ref(dist_features.reshape(M))
    _sc_scatter(upd, tfin.reshape(B), idx, ent_ref, dist_ref)
    new_ent = ent_ref[...]
    new_dist = dist_ref[...]
    bias = (W_entity_b + w_dist_b).reshape(1, 1)
    return _tc_score(hW, new_ent, new_dist.reshape(1, M), mean_t,
                     w_dist_w.reshape(1, 1), bias)
